# baseline (device time: 22070 ns/iter reference)
import os

import jax
import jax.numpy as jnp
from jax import lax
from jax.experimental import pallas as pl
from jax.experimental.pallas import tpu as pltpu

ABLATE = os.environ.get("ABLATE", "")

N_DEV = 4
E_PER = 4
HALF = E_PER // 2
F8 = jnp.float8_e4m3fn
SCALE = 16.0
DQ = 1.0 / SCALE


def kernel(x, router_W, route_idx, expert_W, shared_W):
    m, d = x.shape
    e_loc, _, h = expert_W.shape
    n_exp = router_W.shape[1]

    def body(x_hbm, rw_hbm, idx_hbm, ew_ref, sw_hbm, out_ref,
             myg, grpL, grpR, grpO, xv, rwv, idxv, swv,
             s1, s2, rP1, rO, cp_sems):
        my = lax.axis_index("i")
        left = (my - 1) % N_DEV
        right = (my + 1) % N_DEV

        cps = [
            pltpu.make_async_copy(src, dst, cp_sems.at[i])
            for i, (src, dst) in enumerate(
                [(x_hbm, xv), (rw_hbm, rwv), (idx_hbm, idxv), (sw_hbm, swv)]
            )
        ]
        for cp in cps:
            cp.start()

        if ABLATE != "compute":
            barrier_sem = pltpu.get_barrier_semaphore()
            for nbr in (left, right):
                pl.semaphore_signal(
                    barrier_sem, inc=1,
                    device_id=(nbr,), device_id_type=pl.DeviceIdType.MESH,
                )
            pl.semaphore_wait(barrier_sem, 2)

        myg[...] = (ew_ref[...] * SCALE).astype(F8)

        if ABLATE != "compute":
            p1r_a = pltpu.make_async_remote_copy(
                src_ref=myg.at[pl.ds(HALF, HALF)],
                dst_ref=grpL.at[pl.ds(HALF, HALF)],
                send_sem=s1.at[0], recv_sem=rP1.at[0],
                device_id=(right,), device_id_type=pl.DeviceIdType.MESH,
            )
            p1l_a = pltpu.make_async_remote_copy(
                src_ref=myg.at[pl.ds(0, HALF)],
                dst_ref=grpR.at[pl.ds(0, HALF)],
                send_sem=s1.at[2], recv_sem=rP1.at[2],
                device_id=(left,), device_id_type=pl.DeviceIdType.MESH,
            )
            p1r_b = pltpu.make_async_remote_copy(
                src_ref=myg.at[pl.ds(0, HALF)],
                dst_ref=grpL.at[pl.ds(0, HALF)],
                send_sem=s1.at[1], recv_sem=rP1.at[1],
                device_id=(right,), device_id_type=pl.DeviceIdType.MESH,
            )
            p1l_b = pltpu.make_async_remote_copy(
                src_ref=myg.at[pl.ds(HALF, HALF)],
                dst_ref=grpR.at[pl.ds(HALF, HALF)],
                send_sem=s1.at[3], recv_sem=rP1.at[3],
                device_id=(left,), device_id_type=pl.DeviceIdType.MESH,
            )
            p1r_a.start()
            p1l_a.start()
            p1r_b.start()
            p1l_b.start()

        for cp in cps:
            cp.wait()
        xb = xv[...].astype(jnp.bfloat16)
        scores = jnp.dot(xv[...], rwv[...],
                         preferred_element_type=jnp.float32)
        s_max = jnp.max(scores, axis=1, keepdims=True)
        p = jnp.exp(scores - s_max)
        probs = p / jnp.sum(p, axis=1, keepdims=True)
        idx = idxv[...]
        lane = lax.broadcasted_iota(jnp.int32, (m, n_exp), 1)
        p_routed = jnp.sum(jnp.where(lane == idx, probs, 0.0),
                           axis=1, keepdims=True)

        def accum_group(w_group, origin, off, acc, scale=1.0):
            if ABLATE == "comm":
                return acc
            for j in range(w_group.shape[0]):
                gid = origin * E_PER + off + j
                coef = jnp.where(idx == gid, p_routed, 0.0) * scale
                xm = xb * coef.astype(jnp.bfloat16)
                acc = acc + jnp.dot(xm, w_group[j],
                                    preferred_element_type=jnp.float32)
            return acc

        if ABLATE == "comm":
            acc = jnp.zeros((m, h), jnp.float32)
        else:
            acc = jnp.dot(xb, swv[...].astype(jnp.bfloat16),
                          preferred_element_type=jnp.float32)
        local_w = ew_ref[...].astype(jnp.bfloat16)
        acc = accum_group(local_w, my, 0, acc)

        if ABLATE == "compute":
            for o in range(1, N_DEV):
                acc = accum_group(local_w, (my + o) % N_DEV, 0, acc)
            out_ref[...] = acc.astype(out_ref.dtype)
            return

        p1l_a.wait_recv()
        p2_l = pltpu.make_async_remote_copy(
            src_ref=grpR.at[pl.ds(0, HALF)], dst_ref=grpO.at[pl.ds(0, HALF)],
            send_sem=s2.at[0], recv_sem=rO.at[0],
            device_id=(left,), device_id_type=pl.DeviceIdType.MESH,
        )
        p2_l.start()
        p1r_a.wait_recv()
        p2_r = pltpu.make_async_remote_copy(
            src_ref=grpL.at[pl.ds(HALF, HALF)],
            dst_ref=grpO.at[pl.ds(HALF, HALF)],
            send_sem=s2.at[1], recv_sem=rO.at[1],
            device_id=(right,), device_id_type=pl.DeviceIdType.MESH,
        )
        p2_r.start()

        acc = accum_group(grpR[pl.ds(0, HALF)].astype(jnp.bfloat16),
                          right, 0, acc, DQ)
        acc = accum_group(grpL[pl.ds(HALF, HALF)].astype(jnp.bfloat16),
                          left, HALF, acc, DQ)
        p1l_b.wait_recv()
        acc = accum_group(grpR[pl.ds(HALF, HALF)].astype(jnp.bfloat16),
                          right, HALF, acc, DQ)
        p1r_b.wait_recv()
        acc = accum_group(grpL[pl.ds(0, HALF)].astype(jnp.bfloat16),
                          left, 0, acc, DQ)

        opp = (my + 2) % N_DEV
        p2_l.wait_recv()
        acc = accum_group(grpO[pl.ds(0, HALF)].astype(jnp.bfloat16),
                          opp, 0, acc, DQ)
        p2_r.wait_recv()
        acc = accum_group(grpO[pl.ds(HALF, HALF)].astype(jnp.bfloat16),
                          opp, HALF, acc, DQ)

        for rdma in (p1r_a, p1r_b, p1l_a, p1l_b, p2_l, p2_r):
            rdma.wait_send()
        out_ref[...] = acc.astype(out_ref.dtype)

    return pl.pallas_call(
        body,
        out_shape=jax.ShapeDtypeStruct((m, h), jnp.bfloat16),
        in_specs=[
            pl.BlockSpec(memory_space=pltpu.MemorySpace.HBM),
            pl.BlockSpec(memory_space=pltpu.MemorySpace.HBM),
            pl.BlockSpec(memory_space=pltpu.MemorySpace.HBM),
            pl.BlockSpec(memory_space=pltpu.MemorySpace.VMEM),
            pl.BlockSpec(memory_space=pltpu.MemorySpace.HBM),
        ],
        out_specs=pl.BlockSpec(memory_space=pltpu.MemorySpace.VMEM),
        scratch_shapes=[
            pltpu.VMEM((E_PER, d, h), F8),
            pltpu.VMEM((E_PER, d, h), F8),
            pltpu.VMEM((E_PER, d, h), F8),
            pltpu.VMEM((E_PER, d, h), F8),
            pltpu.VMEM((m, d), jnp.float32),
            pltpu.VMEM((d, n_exp), jnp.float32),
            pltpu.VMEM((m, 1), jnp.int32),
            pltpu.VMEM((d, h), jnp.float32),
            pltpu.SemaphoreType.DMA((4,)),
            pltpu.SemaphoreType.DMA((2,)),
            pltpu.SemaphoreType.DMA((4,)),
            pltpu.SemaphoreType.DMA((2,)),
            pltpu.SemaphoreType.DMA((4,)),
        ],
        compiler_params=(
            pltpu.CompilerParams()
            if ABLATE == "compute"
            else pltpu.CompilerParams(collective_id=0)
        ),
    )(x, router_W, route_idx, expert_W, shared_W)
